# Initial kernel scaffold; baseline (speedup 1.0000x reference)
import jax
import jax.numpy as jnp
from jax.experimental import pallas as pl


def kernel(atomic_numbers, edge_index, W_edge, W_node):
    raise NotImplementedError("write your pallas kernel here")


# speedup vs baseline: 32.2892x; 32.2892x over previous
import functools

import jax
import jax.numpy as jnp
from jax import lax
from jax.experimental import pallas as pl
from jax.experimental.pallas import tpu as pltpu
from jax.experimental.pallas import tpu_sc as plsc

_E = 160000
_N = 10000
_CH = 16
_BDMAX = 14
_ROW = 2 * _CH * _BDMAX

_NW = 32
_EPW = 4992
_TAILW = (_E - _NW * _EPW) // 128
_TAIL0 = _NW * _EPW
_BUF = _EPW + 128

_mesh = plsc.VectorSubcoreMesh(core_axis_name="c", subcore_axis_name="s")


@functools.partial(
    pl.kernel,
    out_type=(jax.ShapeDtypeStruct((1, _E), jnp.int32),
              jax.ShapeDtypeStruct((2, _E), jnp.int32)),
    mesh=_mesh,
    compiler_params=pltpu.CompilerParams(needs_layout_passes=False),
    scratch_types=[
        pltpu.VMEM((2, _BUF), jnp.int32),
        pltpu.VMEM((_N,), jnp.int32),
        pltpu.VMEM((_BUF,), jnp.int32),
        pltpu.SemaphoreType.DMA,
    ],
)
def _sc_pair(ei_hbm, an_hbm, pair_hbm, eio_hbm, ei_v, an_v, pair_v, sem):
    wid = lax.axis_index("s") * 2 + lax.axis_index("c")
    base = wid * _EPW
    tbase = _TAIL0 + wid * 128

    pltpu.sync_copy(ei_hbm.at[:, pl.ds(base, _EPW)], ei_v.at[:, pl.ds(0, _EPW)])

    @pl.when(wid < _TAILW)
    def _load_tail():
        pltpu.sync_copy(ei_hbm.at[:, pl.ds(tbase, 128)],
                        ei_v.at[:, pl.ds(_EPW, 128)])

    eio_cp = pltpu.async_copy(ei_v.at[:, pl.ds(0, _EPW)],
                              eio_hbm.at[:, pl.ds(base, _EPW)], sem)

    pltpu.sync_copy(an_hbm, an_v)

    def body(i, carry):
        s = ei_v[0, pl.ds(i * 16, 16)]
        d = ei_v[1, pl.ds(i * 16, 16)]
        a = plsc.load_gather(an_v, [s])
        b = plsc.load_gather(an_v, [d])
        pair_v[pl.ds(i * 16, 16)] = a * 4 + b
        return carry

    nchunks = _EPW // 16 + jnp.where(wid < _TAILW, 8, 0)
    lax.fori_loop(0, nchunks, body, 0)
    pltpu.sync_copy(pair_v.at[pl.ds(0, _EPW)],
                    pair_hbm.at[0, pl.ds(base, _EPW)])

    @pl.when(wid < _TAILW)
    def _store_tail():
        pltpu.sync_copy(pair_v.at[pl.ds(_EPW, 128)],
                        pair_hbm.at[0, pl.ds(tbase, 128)])
        pltpu.sync_copy(ei_v.at[:, pl.ds(_EPW, 128)],
                        eio_hbm.at[:, pl.ds(tbase, 128)])

    eio_cp.wait()


_B = 16000
_GRID = _E // _B


def _tc_expand_body(pair_ref, w_ref, out_ref, tt_ref):
    @pl.when(pl.program_id(0) == 0)
    def _build_table():
        w = w_ref[...]
        r = lax.broadcasted_iota(jnp.int32, (16, 16), 0)
        j = lax.broadcasted_iota(jnp.int32, (16, 16), 1)
        pswap = (j == (r % 4) * 4 + r // 4).astype(jnp.float32)
        wswap = jnp.dot(pswap, w, preferred_element_type=jnp.float32)
        c = lax.broadcasted_iota(jnp.int32, (_ROW, 48), 0)
        k = lax.broadcasted_iota(jnp.int32, (_ROW, 48), 1)
        sat = (c == (k // 16) * 32 + (k % 16)).astype(jnp.float32)
        sbt = (c == (k // 16) * 32 + 16 + (k % 16)).astype(jnp.float32)
        tta = lax.dot_general(sat, w, (((1,), (1,)), ((), ())),
                              preferred_element_type=jnp.float32)
        ttb = lax.dot_general(sbt, wswap, (((1,), (1,)), ((), ())),
                              preferred_element_type=jnp.float32)
        cc = lax.broadcasted_iota(jnp.int32, (_ROW, 16), 0)
        q = lax.broadcasted_iota(jnp.int32, (_ROW, 16), 1)
        sda = jnp.where(q < 4, 2, 3)
        sdb = jnp.where(q % 4 == 0, 2, 3)
        ma = ((cc % 32) < 16) & ((cc // 32) < sda)
        mb = ((cc % 32) >= 16) & ((cc // 32) < sdb)
        tt_ref[...] = jnp.where(ma, tta, 0.0) + jnp.where(mb, ttb, 0.0)

    cls = lax.broadcasted_iota(jnp.int32, (16, _B), 0)
    oht = (pair_ref[...] == cls).astype(jnp.float32)
    out_ref[...] = jnp.dot(tt_ref[...], oht,
                           preferred_element_type=jnp.float32)


_tc_expand = pl.pallas_call(
    _tc_expand_body,
    grid=(_GRID,),
    in_specs=[
        pl.BlockSpec((1, _B), lambda i: (0, i)),
        pl.BlockSpec((16, 48), lambda i: (0, 0)),
    ],
    out_specs=pl.BlockSpec((_ROW, _B), lambda i: (0, i)),
    out_shape=jax.ShapeDtypeStruct((_ROW, _E), jnp.float32),
    scratch_shapes=[pltpu.VMEM((_ROW, 16), jnp.float32)],
    compiler_params=pltpu.CompilerParams(vmem_limit_bytes=120 * 1024 * 1024),
)


def kernel(atomic_numbers, edge_index, W_edge, W_node):
    del W_node
    an = atomic_numbers.astype(jnp.int32)
    ei = edge_index.astype(jnp.int32)
    pair, ei_out = _sc_pair(ei, an)
    out_t = _tc_expand(pair, W_edge.astype(jnp.float32))
    out = out_t.reshape(_BDMAX, 2 * _CH, _E).transpose(2, 0, 1)
    return out, ei_out
